# Initial kernel scaffold; baseline (speedup 1.0000x reference)
import jax
import jax.numpy as jnp
from jax.experimental import pallas as pl


def kernel(x, router_W, router_b, w_c_fc):
    raise NotImplementedError("write your pallas kernel here")


# speedup vs baseline: 6.2257x; 6.2257x over previous
import jax
import jax.numpy as jnp
from jax.experimental import pallas as pl


_LANES = 128


def _moe_body(x_ref, rw_ref, rb_ref, w_ref, o_ref):
    xb = x_ref[...]
    logits = jnp.dot(xb, rw_ref[...], preferred_element_type=jnp.float32)
    logits = logits + rb_ref[...]
    m = jnp.max(logits, axis=1, keepdims=True)
    lane = jax.lax.broadcasted_iota(jnp.int32, logits.shape, 1)
    eid = jnp.min(jnp.where(logits >= m, lane, _LANES), axis=1, keepdims=True)

    E = w_ref.shape[0]
    acc = jnp.zeros(o_ref.shape, dtype=jnp.float32)
    for e in range(E):
        mask = (eid == e).astype(xb.dtype)
        acc = acc + jnp.dot(xb * mask, w_ref[e], preferred_element_type=jnp.float32)
    o_ref[...] = acc


def kernel(x, router_W, router_b, w_c_fc):
    B, T, C = x.shape
    E = w_c_fc.shape[0]
    N = B * T
    TM = 512
    x2 = x.reshape(N, C)

    rw = jnp.zeros((C, _LANES), jnp.float32).at[:, :E].set(router_W)
    rb = jnp.full((1, _LANES), -jnp.inf, jnp.float32).at[0, :E].set(router_b)

    y2 = pl.pallas_call(
        _moe_body,
        grid=(N // TM,),
        in_specs=[
            pl.BlockSpec((TM, C), lambda i: (i, 0)),
            pl.BlockSpec((C, _LANES), lambda i: (0, 0)),
            pl.BlockSpec((1, _LANES), lambda i: (0, 0)),
            pl.BlockSpec((E, C, C), lambda i: (0, 0, 0)),
        ],
        out_specs=pl.BlockSpec((TM, C), lambda i: (i, 0)),
        out_shape=jax.ShapeDtypeStruct((N, C), jnp.float32),
    )(x2, rw, rb, w_c_fc)
    return y2.reshape(B, T, C)
